# baseline (speedup 1.0000x reference)
import functools

import jax
import jax.numpy as jnp
from jax import lax
from jax.experimental import pallas as pl
from jax.experimental.pallas import tpu as pltpu
from jax.experimental.pallas import tpu_sc as plsc

_NC, _NS, _L = 2, 16, 16
_NW = _NC * _NS

_E0 = 320000
_N0 = 100000
_E1 = 65536
_N1 = 20000
_ND0 = 20000
_ND1 = 4096
_K = 128


def _hist_phase(w, src_hbm, out_hbm, chunk_v, hist_v, E, B, chunk_len, nchunks):
    ones = jnp.ones((_L,), jnp.float32)
    base = w * (E // _NW)

    def zero_hist(i, _):
        hist_v[pl.ds(i * _L, _L)] = jnp.zeros((_L,), jnp.float32)
        return 0
    lax.fori_loop(0, B // _L, zero_hist, 0)

    for k in range(nchunks):
        pltpu.sync_copy(src_hbm.at[pl.ds(base + k * chunk_len, chunk_len)],
                        chunk_v.at[pl.ds(0, chunk_len)])

        def scan(i, _):
            idx = chunk_v[pl.ds(i * _L, _L)]
            plsc.addupdate_scatter(hist_v, [idx], ones)
            return 0
        lax.fori_loop(0, chunk_len // _L, scan, 0)

    pltpu.sync_copy(hist_v.at[pl.ds(0, B)], out_hbm.at[pl.ds(w * B, B)])


def _hist_body(src0_hbm, src1_hbm, d0_hbm, d1_hbm, chunk_v, hist_v):
    c = lax.axis_index("c")
    s = lax.axis_index("s")
    w = s * _NC + c
    _hist_phase(w, src0_hbm, d0_hbm, chunk_v, hist_v, _E0, _N0, 2000, 5)
    _hist_phase(w, src1_hbm, d1_hbm, chunk_v, hist_v, _E1, _N1, 2048, 1)


_hist_call = functools.partial(
    pl.kernel,
    _hist_body,
    out_type=[
        jax.ShapeDtypeStruct((_NW * _N0,), jnp.float32),
        jax.ShapeDtypeStruct((_NW * _N1,), jnp.float32),
    ],
    mesh=plsc.VectorSubcoreMesh(core_axis_name="c", subcore_axis_name="s"),
    scratch_types=[
        pltpu.VMEM((2048,), jnp.int32),
        pltpu.VMEM((_N0,), jnp.float32),
    ],
    compiler_params=pltpu.CompilerParams(needs_layout_passes=False),
    name="sc_degree_hists",
)


def _make_spmm(E, n_table, per_core, chunk, nchunks, name):
    e_per_tile = E // _NS
    assert e_per_tile == chunk * nchunks
    dump = per_core
    acc_rows = ((per_core + 16 + 127) // 128) * 128
    stripe = acc_rows // _NS
    out_stripe = (per_core // _NS) // 8 * 8
    out_rem = per_core - out_stripe * _NS
    nb_max = chunk // _K

    def body(src_hbm, dst_hbm, table_hbm, out_hbm, hist_hbm,
             srcv, dstv, csrc, cdlf, cdl2, rows, histv, acc_sh, sem):
        c = lax.axis_index("c")
        s = lax.axis_index("s")
        w = s * _NC + c
        lo = c * per_core
        zero16 = jnp.zeros((_L,), jnp.float32)
        ones16 = jnp.ones((_L,), jnp.float32)

        def zrow(r, _):
            for u in range(8):
                rows[r, pl.ds(u * _L, _L)] = zero16
            return 0
        lax.fori_loop(0, _K, zrow, 0)
        for off in range(0, stripe, _K):
            n = min(_K, stripe - off)
            pltpu.sync_copy(rows.at[pl.ds(0, n), :],
                            acc_sh.at[pl.ds(s * stripe + off, n), :])

        def zhist(i, _):
            histv[pl.ds(i * _L, _L)] = zero16
            return 0
        lax.fori_loop(0, (per_core + 16) // _L, zhist, 0)
        plsc.subcore_barrier()

        for k in range(nchunks):
            base_e = s * e_per_tile + k * chunk
            pltpu.sync_copy(src_hbm.at[pl.ds(base_e, chunk)],
                            srcv.at[pl.ds(0, chunk)])
            pltpu.sync_copy(dst_hbm.at[pl.ds(base_e, chunk)],
                            dstv.at[pl.ds(0, chunk)])

            def pre(i, _):
                csrc[pl.ds(i * _L, _L)] = jnp.zeros((_L,), jnp.int32)
                cdlf[pl.ds(i * _L, _L)] = jnp.full((_L,), dump, jnp.int32)
                return 0
            lax.fori_loop(0, chunk // _L, pre, 0)

            def scan(i, cnt):
                d16 = dstv[pl.ds(i * _L, _L)]
                s16 = srcv[pl.ds(i * _L, _L)]
                m = (d16 >= lo) & (d16 < lo + per_core)
                dl = d16 - lo
                plsc.store_compressed(csrc.at[pl.ds(cnt, _L)], s16, mask=m)
                plsc.store_compressed(cdlf.at[pl.ds(cnt, _L)], dl, mask=m)
                dl_h = jnp.where(m, dl, per_core)
                plsc.addupdate_scatter(histv, [dl_h], ones16, mask=m)
                return cnt + jnp.sum(m.astype(jnp.int32))
            cnt = lax.fori_loop(0, chunk // _L, scan, 0)

            def tocdl2(j, _):
                cdl2[j // 8, pl.ds((j % 8) * _L, _L)] = cdlf[pl.ds(j * _L, _L)]
                return 0
            lax.fori_loop(0, chunk // _L, tocdl2, 0)

            nb = (cnt + _K - 1) // _K

            def batch(b, _):
                pltpu.async_copy(table_hbm.at[csrc.at[pl.ds(b * _K, _K)]],
                                 rows, sem).wait()
                pltpu.sync_copy(rows, acc_sh.at[cdl2.at[b]], add=True)
                return 0
            lax.fori_loop(0, nb, batch, 0)

        plsc.subcore_barrier()

        for off in range(0, out_stripe, _K):
            n = min(_K, out_stripe - off)
            pltpu.sync_copy(acc_sh.at[pl.ds(s * out_stripe + off, n), :],
                            out_hbm.at[pl.ds(c * per_core + s * out_stripe + off, n), :])
        if out_rem:
            @pl.when(s == 0)
            def _copy_rem():
                pltpu.sync_copy(
                    acc_sh.at[pl.ds(out_stripe * _NS, out_rem), :],
                    out_hbm.at[pl.ds(c * per_core + out_stripe * _NS, out_rem), :])
        pltpu.sync_copy(histv.at[pl.ds(0, per_core)],
                        hist_hbm.at[pl.ds(w * per_core, per_core)])

    return functools.partial(
        pl.kernel,
        body,
        out_type=[
            jax.ShapeDtypeStruct((_NC * per_core, 128), jnp.float32),
            jax.ShapeDtypeStruct((_NW * per_core,), jnp.float32),
        ],
        mesh=plsc.VectorSubcoreMesh(core_axis_name="c", subcore_axis_name="s"),
        scratch_types=[
            pltpu.VMEM((chunk,), jnp.int32),
            pltpu.VMEM((chunk,), jnp.int32),
            pltpu.VMEM((chunk,), jnp.int32),
            pltpu.VMEM((chunk,), jnp.int32),
            pltpu.VMEM((nb_max, _K), jnp.int32),
            pltpu.VMEM((_K, 128), jnp.float32),
            pltpu.VMEM((per_core + 16,), jnp.float32),
            pltpu.VMEM_SHARED((acc_rows, 128), jnp.float32),
            pltpu.SemaphoreType.DMA,
        ],
        compiler_params=pltpu.CompilerParams(needs_layout_passes=False),
        name=name,
    )


_spmm0_call = _make_spmm(_E0, _N0, _ND0 // 2, 2000, 10, "sc_spmm0")
_spmm1_call = _make_spmm(_E1, _N1, _ND1 // 2, 2048, 2, "sc_spmm1")


def _tc1_body(x_ref, p_ref, o_ref):
    deg = jnp.sum(p_ref[...], axis=(0, 1, 2))
    norm = lax.rsqrt(jnp.clip(deg, 1.0, None))
    o_ref[...] = x_ref[...] * norm[:, None]


def _tc1(x, d0p):
    return pl.pallas_call(
        _tc1_body,
        grid=(125,),
        in_specs=[
            pl.BlockSpec((800, 128), lambda i: (i, 0)),
            pl.BlockSpec((_NW, 1, 1, 800), lambda i: (0, i, 0, 0)),
        ],
        out_specs=pl.BlockSpec((800, 128), lambda i: (i, 0)),
        out_shape=jax.ShapeDtypeStruct((_N0, 128), jnp.float32),
    )(x, d0p.reshape(_NW, 125, 1, 800))


def _tc2_body(agg_ref, hd0_ref, d1p_ref, w1_ref, b1_ref, w2_ref, o_ref):
    degd = jnp.sum(hd0_ref[...], axis=(0, 1, 2, 3))
    z = agg_ref[...] @ w1_ref[...]
    z = z * lax.rsqrt(jnp.clip(degd, 1.0, None))[:, None] + b1_ref[...]
    z = jax.nn.relu(z)
    degs = jnp.sum(d1p_ref[...], axis=(0, 1, 2))
    z = z * lax.rsqrt(jnp.clip(degs, 1.0, None))[:, None]
    t2 = z @ w2_ref[...]
    o_ref[...] = jnp.concatenate(
        [t2, jnp.zeros((t2.shape[0], 64), jnp.float32)], axis=1)


def _tc2(agg0, hd0, d1p, W1, b1, W2):
    return pl.pallas_call(
        _tc2_body,
        grid=(50,),
        in_specs=[
            pl.BlockSpec((400, 128), lambda i: (i, 0)),
            pl.BlockSpec((_NS, 1, 1, 1, 400), lambda i: (0, i // 25, i % 25, 0, 0)),
            pl.BlockSpec((_NW, 1, 1, 400), lambda i: (0, i, 0, 0)),
            pl.BlockSpec((128, 128), lambda i: (0, 0)),
            pl.BlockSpec((1, 128), lambda i: (0, 0)),
            pl.BlockSpec((128, 64), lambda i: (0, 0)),
        ],
        out_specs=pl.BlockSpec((400, 128), lambda i: (i, 0)),
        out_shape=jax.ShapeDtypeStruct((_N1, 128), jnp.float32),
    )(agg0, hd0.reshape(_NS, _NC, 25, 1, 400), d1p.reshape(_NW, 50, 1, 400),
      W1, b1.reshape(1, 128), W2)


def _tc3_body(agg_ref, hd1_ref, b2_ref, o_ref):
    degd = jnp.sum(hd1_ref[...], axis=(0, 1, 2, 3))
    norm = lax.rsqrt(jnp.clip(degd, 1.0, None))
    o_ref[...] = agg_ref[:, :64] * norm[:, None] + b2_ref[...]


def _tc3(agg1, hd1, b2):
    return pl.pallas_call(
        _tc3_body,
        grid=(8,),
        in_specs=[
            pl.BlockSpec((512, 128), lambda i: (i, 0)),
            pl.BlockSpec((_NS, 1, 1, 1, 512), lambda i: (0, i // 4, i % 4, 0, 0)),
            pl.BlockSpec((1, 64), lambda i: (0, 0)),
        ],
        out_specs=pl.BlockSpec((512, 64), lambda i: (i, 0)),
        out_shape=jax.ShapeDtypeStruct((_ND1, 64), jnp.float32),
    )(agg1, hd1.reshape(_NS, _NC, 4, 1, 512), b2.reshape(1, 64))


def kernel(x, mfg0_src, mfg0_dst, mfg0_num_dst, mfg1_src, mfg1_dst, mfg1_num_dst, W1, b1, W2, b2):
    d0p, d1p = _hist_call()(mfg0_src, mfg1_src)
    h = _tc1(x, d0p)
    agg0, hd0 = _spmm0_call()(mfg0_src, mfg0_dst, h)
    t2p = _tc2(agg0, hd0, d1p, W1, b1, W2)
    agg1, hd1 = _spmm1_call()(mfg1_src, mfg1_dst, t2p)
    return _tc3(agg1, hd1, b2)
